# baseline (speedup 1.0000x reference)
import functools

import jax
import jax.numpy as jnp
from jax import lax
from jax.experimental import pallas as pl
from jax.experimental.pallas import tpu as pltpu
from jax.experimental.pallas import tpu_sc as plsc

_NC = 2
_NS = 16
_NW = _NC * _NS
_L = 16


def _transe_sc(h_head, h_tail, rels, rel_emb):
    B, F = h_head.shape
    bpw = B // _NW
    C = min(bpw, 128)
    nchunks = bpw // C
    mesh = plsc.VectorSubcoreMesh(core_axis_name="c", subcore_axis_name="s")

    @functools.partial(
        pl.kernel,
        out_type=jax.ShapeDtypeStruct((B,), jnp.float32),
        mesh=mesh,
        scratch_types=dict(
            idx_v=pltpu.VMEM((bpw,), jnp.int32),
            head_v=pltpu.VMEM((C, F), jnp.float32),
            tail_v=pltpu.VMEM((C, F), jnp.float32),
            rel_v=pltpu.VMEM((C, F), jnp.float32),
            out_v=pltpu.VMEM((bpw,), jnp.float32),
            sem=pltpu.SemaphoreType.DMA,
        ),
        compiler_params=pltpu.CompilerParams(needs_layout_passes=False),
    )
    def k(head_hbm, tail_hbm, rels_hbm, emb_hbm, out_hbm,
          idx_v, head_v, tail_v, rel_v, out_v, sem):
        wid = lax.axis_index("s") * _NC + lax.axis_index("c")
        base = wid * bpw
        pltpu.sync_copy(rels_hbm.at[pl.ds(base, bpw)], idx_v)
        lanes = lax.iota(jnp.int32, _L)
        for g in range(nchunks):
            row0 = base + g * C
            cp_h = pltpu.async_copy(head_hbm.at[pl.ds(row0, C)], head_v, sem)
            cp_t = pltpu.async_copy(tail_hbm.at[pl.ds(row0, C)], tail_v, sem)
            cp_r = pltpu.async_copy(emb_hbm.at[idx_v.at[pl.ds(g * C, C)]],
                                    rel_v, sem)
            cp_h.wait()
            cp_t.wait()
            cp_r.wait()

            def group_body(g2, carry, g=g):
                out_acc = jnp.zeros((_L,), jnp.float32)
                for r2 in range(_L):
                    row = g2 * _L + r2
                    acc = jnp.zeros((_L,), jnp.float32)
                    for v in range(F // _L):
                        hv = head_v[row, pl.ds(v * _L, _L)]
                        tv = tail_v[row, pl.ds(v * _L, _L)]
                        rv = rel_v[row, pl.ds(v * _L, _L)]
                        acc = acc + jnp.abs(hv + rv - tv)
                    s = jnp.sum(acc)
                    out_acc = jnp.where(lanes == r2, -s, out_acc)
                out_v[pl.ds(g * C + g2 * _L, _L)] = out_acc
                return carry

            lax.fori_loop(0, C // _L, group_body, 0)
        pltpu.sync_copy(out_v, out_hbm.at[pl.ds(base, bpw)])

    return k(h_head, h_tail, rels, rel_emb)


def kernel(h_head, h_tail, rels, rel_emb):
    return _transe_sc(h_head, h_tail, rels.astype(jnp.int32), rel_emb)
